# baseline (speedup 1.0000x reference)
import jax
import jax.numpy as jnp
from jax.experimental import pallas as pl

_B, _L, _D, _V = 64, 1024, 128, 33
_LANES = 1024
_ROWS = 8
_KPAD = 56


def _inf0(x):
    return jnp.where(jnp.isinf(x), 0.0, x)


_SIN_C = (6.283183466376198, -41.34148035624613, 81.59765787614148,
          -76.59492821657112, 41.269929567669145, -12.372494818439662)
_COS_C = (0.999999992290297, -19.73920555404448, 64.93917223259542,
          -85.45116579292082, 60.176230338868066, -26.000527873748382,
          6.575611642718274)
_TWO_PI_INV = 0.15915494309189535


def _sincos(a):
    r = a * _TWO_PI_INV
    r = r - jnp.round(r)
    s = r * r
    sp = _SIN_C[-1]
    for coef in _SIN_C[-2::-1]:
        sp = sp * s + coef
    cp = _COS_C[-1]
    for coef in _COS_C[-2::-1]:
        cp = cp * s + coef
    return cp, sp * r


def _body(r_ref, mseq_ref, mstr_ref, bl_ref, ang_ref, w_ref, out_ref):
    rows = r_ref.shape[0]
    lanes = r_ref.shape[1]
    mstr = mstr_ref[...]
    notm = (1.0 - mstr)[:, None, :]

    blv = _inf0(bl_ref[...]) * notm
    cos_a, sin_a = _sincos(_inf0(ang_ref[...]))
    cos_a = cos_a * notm
    sin_a = sin_a * notm

    idx = jnp.where(mseq_ref[...] != 0, _V, r_ref[...])
    iota = jax.lax.broadcasted_iota(jnp.int32, (rows, 40, lanes), 1)
    onehot = jnp.where(idx[:, None, :] == iota, 1.0, 0.0)
    c = jnp.concatenate([onehot, blv, cos_a, sin_a, mstr[:, None, :]],
                        axis=1).astype(jnp.bfloat16)
    res = jax.lax.dot_general(
        c, w_ref[...].astype(jnp.bfloat16), (((1,), (0,)), ((), ())),
        preferred_element_type=jnp.float32)
    out_ref[...] = res


def kernel(R, bl_N_CA, bl_CA_C, bl_C_N,
           ba_C_N_CA, ba_N_CA_C, ba_CA_C_N,
           da_CA_C_N_CA, da_C_N_CA_C, da_N_CA_C_N,
           mask_seq, mask_str,
           res_table, bl_W, ba_W, da_W, seq_mask_w, str_mask_w):
    nrows = _B * _L // _LANES
    rs = lambda x: x.reshape(nrows, _LANES)

    baT = ba_W.T / 3.0
    daT = da_W.T / 3.0
    wbig = jnp.concatenate([
        res_table,
        seq_mask_w,
        jnp.zeros((6, _D), jnp.float32),
        bl_W.T / 3.0,
        baT[0::2], daT[0::2],
        baT[1::2], daT[1::2],
        str_mask_w,
    ], axis=0)

    bl3 = jnp.stack([rs(bl_N_CA), rs(bl_CA_C), rs(bl_C_N)], axis=1)
    ang6 = jnp.stack([rs(ba_C_N_CA), rs(ba_N_CA_C), rs(ba_CA_C_N),
                      rs(da_CA_C_N_CA), rs(da_C_N_CA_C), rs(da_N_CA_C_N)],
                     axis=1)

    ins = [
        rs(R.astype(jnp.int32)),
        rs(mask_seq.astype(jnp.int32)),
        rs(mask_str.astype(jnp.float32)),
        bl3,
        ang6,
        wbig,
    ]

    tok_spec = pl.BlockSpec((_ROWS, _LANES), lambda g: (g, 0))
    in_specs = [
        tok_spec, tok_spec, tok_spec,
        pl.BlockSpec((_ROWS, 3, _LANES), lambda g: (g, 0, 0)),
        pl.BlockSpec((_ROWS, 6, _LANES), lambda g: (g, 0, 0)),
        pl.BlockSpec((_KPAD, _D), lambda g: (0, 0)),
    ]
    out = pl.pallas_call(
        _body,
        grid=(nrows // _ROWS,),
        in_specs=in_specs,
        out_specs=pl.BlockSpec((_ROWS, _LANES, _D), lambda g: (g, 0, 0)),
        out_shape=jax.ShapeDtypeStruct((nrows, _LANES, _D), jnp.float32),
    )(*ins)
    return out.reshape(_B, _L, _D)


# speedup vs baseline: 1.6272x; 1.6272x over previous
import jax
import jax.numpy as jnp
from jax.experimental import pallas as pl

_B, _L, _D, _V = 64, 1024, 128, 33
_LANES = 1024
_ROWS = 8
_KPAD = 56


def _inf0(x):
    return jnp.where(jnp.isinf(x), 0.0, x)


_SIN_C = (6.283183466376198, -41.34148035624613, 81.59765787614148,
          -76.59492821657112, 41.269929567669145, -12.372494818439662)
_COS_C = (0.999999992290297, -19.73920555404448, 64.93917223259542,
          -85.45116579292082, 60.176230338868066, -26.000527873748382,
          6.575611642718274)
_TWO_PI_INV = 0.15915494309189535


def _sincos(a):
    r = a * _TWO_PI_INV
    r = r - jnp.round(r)
    s = r * r
    sp = _SIN_C[-1]
    for coef in _SIN_C[-2::-1]:
        sp = sp * s + coef
    cp = _COS_C[-1]
    for coef in _COS_C[-2::-1]:
        cp = cp * s + coef
    return cp, sp * r


def _body(r_ref, mseq_ref, mstr_ref,
          bl0_ref, bl1_ref, bl2_ref,
          a0_ref, a1_ref, a2_ref,
          d0_ref, d1_ref, d2_ref,
          w_ref, out_ref):
    rows = r_ref.shape[0]
    lanes = r_ref.shape[1]
    mstr = mstr_ref[...]
    notm = 1.0 - mstr
    feats = [_inf0(bl0_ref[...]), _inf0(bl1_ref[...]), _inf0(bl2_ref[...])]
    for a_ref in (a0_ref, a1_ref, a2_ref, d0_ref, d1_ref, d2_ref):
        cos_a, sin_a = _sincos(_inf0(a_ref[...]))
        feats.append(cos_a)
        feats.append(sin_a)
    feats = [f * notm for f in feats]
    feats.append(mstr)

    idx = jnp.where(mseq_ref[...] != 0, _V, r_ref[...])
    iota = jax.lax.broadcasted_iota(jnp.int32, (rows, 40, lanes), 1)
    onehot = jnp.where(idx[:, None, :] == iota, 1.0, 0.0)
    c = jnp.concatenate([onehot] + [f[:, None, :] for f in feats],
                        axis=1).astype(jnp.bfloat16)
    res = jax.lax.dot_general(
        c, w_ref[...].astype(jnp.bfloat16), (((1,), (0,)), ((), ())),
        preferred_element_type=jnp.float32)
    out_ref[...] = res.reshape(rows * lanes, _D)


def kernel(R, bl_N_CA, bl_CA_C, bl_C_N,
           ba_C_N_CA, ba_N_CA_C, ba_CA_C_N,
           da_CA_C_N_CA, da_C_N_CA_C, da_N_CA_C_N,
           mask_seq, mask_str,
           res_table, bl_W, ba_W, da_W, seq_mask_w, str_mask_w):
    nrows = _B * _L // _LANES
    rs = lambda x: x.reshape(nrows, _LANES)

    wbig = jnp.concatenate([
        res_table,
        seq_mask_w,
        jnp.zeros((6, _D), jnp.float32),
        bl_W.T / 3.0,
        ba_W.T / 3.0,
        da_W.T / 3.0,
        str_mask_w,
    ], axis=0)

    ins = [
        rs(R.astype(jnp.int32)),
        rs(mask_seq.astype(jnp.int32)),
        rs(mask_str.astype(jnp.float32)),
        rs(bl_N_CA), rs(bl_CA_C), rs(bl_C_N),
        rs(ba_C_N_CA), rs(ba_N_CA_C), rs(ba_CA_C_N),
        rs(da_CA_C_N_CA), rs(da_C_N_CA_C), rs(da_N_CA_C_N),
        wbig,
    ]

    tok_spec = pl.BlockSpec((_ROWS, _LANES), lambda g: (g, 0))
    in_specs = [tok_spec] * 12 + [pl.BlockSpec((_KPAD, _D), lambda g: (0, 0))]
    out = pl.pallas_call(
        _body,
        grid=(nrows // _ROWS,),
        in_specs=in_specs,
        out_specs=pl.BlockSpec((_ROWS * _LANES, _D), lambda g: (g, 0)),
        out_shape=jax.ShapeDtypeStruct((_B * _L, _D), jnp.float32),
    )(*ins)
    return out.reshape(_B, _L, _D)
